# baseline (speedup 1.0000x reference)
import functools

import jax
import jax.numpy as jnp
from jax import lax
from jax.experimental import pallas as pl
from jax.experimental.pallas import tpu as pltpu
from jax.experimental.pallas import tpu_sc as plsc

_B = 16384
_D = 64
_IDX_CHUNK = 128


def _make_sc_gather():
    info = plsc.get_sparse_core_info()
    nc, ns = info.num_cores, info.num_subcores
    nw = nc * ns
    b_per_w = _B // nw
    n_chunks = b_per_w // _IDX_CHUNK
    mesh = plsc.VectorSubcoreMesh(core_axis_name="c", subcore_axis_name="s")

    @functools.partial(
        pl.kernel,
        out_type=(
            jax.ShapeDtypeStruct((_B, _D), jnp.float32),
            jax.ShapeDtypeStruct((_B, _D), jnp.float32),
        ),
        mesh=mesh,
        compiler_params=pltpu.CompilerParams(use_tc_tiling_on_sc=False),
        scratch_types=[
            pltpu.VMEM((n_chunks, _IDX_CHUNK), jnp.int32),
            pltpu.VMEM((n_chunks, _IDX_CHUNK), jnp.int32),
            pltpu.VMEM((b_per_w, _D), jnp.float32),
            pltpu.VMEM((b_per_w, _D), jnp.float32),
            pltpu.SemaphoreType.DMA,
            pltpu.SemaphoreType.DMA,
        ],
    )
    def gather_kernel(user_hbm, item_hbm, utab_hbm, itab_hbm, uout_hbm,
                      iout_hbm, uidx_v, iidx_v, urows_v, irows_v, usem, isem):
        wid = lax.axis_index("s") * nc + lax.axis_index("c")
        base = wid * b_per_w
        for j in range(n_chunks):
            pltpu.sync_copy(user_hbm.at[pl.ds(base + j * _IDX_CHUNK, _IDX_CHUNK)],
                            uidx_v.at[j])
            pltpu.sync_copy(item_hbm.at[pl.ds(base + j * _IDX_CHUNK, _IDX_CHUNK)],
                            iidx_v.at[j])
        copies = []
        for j in range(n_chunks):
            copies.append(pltpu.async_copy(
                utab_hbm.at[uidx_v.at[j]],
                urows_v.at[pl.ds(j * _IDX_CHUNK, _IDX_CHUNK), :], usem))
            copies.append(pltpu.async_copy(
                itab_hbm.at[iidx_v.at[j]],
                irows_v.at[pl.ds(j * _IDX_CHUNK, _IDX_CHUNK), :], isem))
        for c in copies:
            c.wait()
        pltpu.sync_copy(urows_v, uout_hbm.at[pl.ds(base, b_per_w)])
        pltpu.sync_copy(irows_v, iout_hbm.at[pl.ds(base, b_per_w)])

    return gather_kernel


_sc_gather_cache = None


def _sc_gather(*args):
    global _sc_gather_cache
    if _sc_gather_cache is None:
        _sc_gather_cache = _make_sc_gather()
    return _sc_gather_cache(*args)

_BLK = 512


def _mlp_body(u_ref, v_ref, w1u_ref, w1v_ref, b1_ref, w2_ref, b2_ref, w3_ref,
              b3_ref, o_ref):
    h = jnp.dot(u_ref[...], w1u_ref[...], preferred_element_type=jnp.float32)
    h = h + jnp.dot(v_ref[...], w1v_ref[...],
                    preferred_element_type=jnp.float32)
    h = jnp.maximum(h + b1_ref[...], 0.0)
    h2 = jnp.dot(h, w2_ref[...], preferred_element_type=jnp.float32)
    h2 = jnp.maximum(h2 + b2_ref[...], 0.0)
    o = jnp.sum(h2 * w3_ref[...], axis=1, keepdims=True) + b3_ref[...]
    o_ref[...] = jax.nn.sigmoid(o)


def _mlp(u, v, w1u, w1v, b1, w2t, b2, w3r, b3):
    grid = _B // _BLK
    full = lambda i: (0, 0)
    return pl.pallas_call(
        _mlp_body,
        grid=(grid,),
        in_specs=[
            pl.BlockSpec((_BLK, _D), lambda i: (i, 0)),
            pl.BlockSpec((_BLK, _D), lambda i: (i, 0)),
            pl.BlockSpec((_D, 256), full),
            pl.BlockSpec((_D, 256), full),
            pl.BlockSpec((1, 256), full),
            pl.BlockSpec((256, 128), full),
            pl.BlockSpec((1, 128), full),
            pl.BlockSpec((1, 128), full),
            pl.BlockSpec((1, 1), full),
        ],
        out_specs=pl.BlockSpec((_BLK, 1), lambda i: (i, 0)),
        out_shape=jax.ShapeDtypeStruct((_B, 1), jnp.float32),
    )(u, v, w1u, w1v, b1, w2t, b2, w3r, b3)


def kernel(user, item, user_table, item_table, W1, b1, W2, b2, W3, b3):
    u_emb, i_emb = _sc_gather(user, item, user_table, item_table)
    w1u = W1[:, :_D].T
    w1v = W1[:, _D:].T
    out = _mlp(u_emb, i_emb, w1u, w1v, b1.reshape(1, 256), W2.T,
               b2.reshape(1, 128), W3, b3.reshape(1, 1))
    return out[:, 0]


# speedup vs baseline: 1.6364x; 1.6364x over previous
import functools

import jax
import jax.numpy as jnp
from jax import lax
from jax.experimental import pallas as pl
from jax.experimental.pallas import tpu as pltpu
from jax.experimental.pallas import tpu_sc as plsc

_B = 16384
_D = 64
_N = 1000000
_IDX_CHUNK = 128
_TCOL = 2048


def _repack_body(a_ref, b_ref, o_ref):
    o_ref[...] = jnp.concatenate([a_ref[...].T, b_ref[...].T], axis=1)


def _repack(utab_t, itab_t):
    grid = (_N + _TCOL - 1) // _TCOL
    return pl.pallas_call(
        _repack_body,
        grid=(grid,),
        in_specs=[
            pl.BlockSpec((_D, _TCOL), lambda i: (0, i)),
            pl.BlockSpec((_D, _TCOL), lambda i: (0, i)),
        ],
        out_specs=pl.BlockSpec((_TCOL, 2 * _D), lambda i: (i, 0)),
        out_shape=jax.ShapeDtypeStruct((_N, 2 * _D), jnp.float32),
    )(utab_t, itab_t)


def _make_sc_gather():
    info = plsc.get_sparse_core_info()
    nc, ns = info.num_cores, info.num_subcores
    nw = nc * ns
    b_per_w = _B // nw
    n_chunks = b_per_w // _IDX_CHUNK
    mesh = plsc.VectorSubcoreMesh(core_axis_name="c", subcore_axis_name="s")

    @functools.partial(
        pl.kernel,
        out_type=(
            jax.ShapeDtypeStruct((_B, 2 * _D), jnp.float32),
            jax.ShapeDtypeStruct((_B, 2 * _D), jnp.float32),
        ),
        mesh=mesh,
        compiler_params=pltpu.CompilerParams(use_tc_tiling_on_sc=True),
        scratch_types=[
            pltpu.VMEM((n_chunks, _IDX_CHUNK), jnp.int32),
            pltpu.VMEM((n_chunks, _IDX_CHUNK), jnp.int32),
            pltpu.VMEM((b_per_w, 2 * _D), jnp.float32),
            pltpu.SemaphoreType.DMA,
        ],
    )
    def gather_kernel(uid_hbm, iid_hbm, tab_hbm, uout_hbm, iout_hbm, uidx_v,
                      iidx_v, rows_v, sem):
        wid = lax.axis_index("s") * nc + lax.axis_index("c")
        base = wid * b_per_w
        for j in range(n_chunks):
            pltpu.sync_copy(uid_hbm.at[pl.ds(base + j * _IDX_CHUNK,
                                             _IDX_CHUNK)], uidx_v.at[j])
            pltpu.sync_copy(iid_hbm.at[pl.ds(base + j * _IDX_CHUNK,
                                             _IDX_CHUNK)], iidx_v.at[j])
        for idx_v, out_hbm in ((uidx_v, uout_hbm), (iidx_v, iout_hbm)):
            copies = []
            for j in range(n_chunks):
                copies.append(pltpu.async_copy(
                    tab_hbm.at[idx_v.at[j]],
                    rows_v.at[pl.ds(j * _IDX_CHUNK, _IDX_CHUNK), :], sem))
            for c in copies:
                c.wait()
            pltpu.sync_copy(rows_v, out_hbm.at[pl.ds(base, b_per_w)])

    return gather_kernel


_sc_gather_cache = None


def _sc_gather(*args):
    global _sc_gather_cache
    if _sc_gather_cache is None:
        _sc_gather_cache = _make_sc_gather()
    return _sc_gather_cache(*args)


_BLK = 512


def _mlp_body(ur_ref, ir_ref, w1u_ref, w1v_ref, b1_ref, w2_ref, b2_ref,
              w3_ref, b3_ref, o_ref):
    u = ur_ref[:, :_D]
    v = ir_ref[:, _D:]
    h = jnp.dot(u, w1u_ref[...], preferred_element_type=jnp.float32)
    h = h + jnp.dot(v, w1v_ref[...], preferred_element_type=jnp.float32)
    h = jnp.maximum(h + b1_ref[...], 0.0)
    h2 = jnp.dot(h, w2_ref[...], preferred_element_type=jnp.float32)
    h2 = jnp.maximum(h2 + b2_ref[...], 0.0)
    o = jnp.sum(h2 * w3_ref[...], axis=1, keepdims=True) + b3_ref[...]
    o_ref[...] = jax.nn.sigmoid(o)


def _mlp(ur, ir, w1u, w1v, b1, w2t, b2, w3r, b3):
    grid = _B // _BLK
    full = lambda i: (0, 0)
    return pl.pallas_call(
        _mlp_body,
        grid=(grid,),
        in_specs=[
            pl.BlockSpec((_BLK, 2 * _D), lambda i: (i, 0)),
            pl.BlockSpec((_BLK, 2 * _D), lambda i: (i, 0)),
            pl.BlockSpec((_D, 256), full),
            pl.BlockSpec((_D, 256), full),
            pl.BlockSpec((1, 256), full),
            pl.BlockSpec((256, 128), full),
            pl.BlockSpec((1, 128), full),
            pl.BlockSpec((1, 128), full),
            pl.BlockSpec((1, 1), full),
        ],
        out_specs=pl.BlockSpec((_BLK, 1), lambda i: (i, 0)),
        out_shape=jax.ShapeDtypeStruct((_B, 1), jnp.float32),
    )(ur, ir, w1u, w1v, b1, w2t, b2, w3r, b3)


def kernel(user, item, user_table, item_table, W1, b1, W2, b2, W3, b3):
    packed = _repack(user_table.T, item_table.T)
    u_rows, i_rows = _sc_gather(user, item, packed)
    w1u = W1[:, :_D].T
    w1v = W1[:, _D:].T
    out = _mlp(u_rows, i_rows, w1u, w1v, b1.reshape(1, 256), W2.T,
               b2.reshape(1, 128), W3, b3.reshape(1, 1))
    return out[:, 0]


# speedup vs baseline: 2.2585x; 1.3801x over previous
import functools

import jax
import jax.numpy as jnp
from jax import lax
from jax.experimental import pallas as pl
from jax.experimental.pallas import tpu as pltpu
from jax.experimental.pallas import tpu_sc as plsc

_B = 16384
_D = 64
_N = 1000000
_IDX_CHUNK = 128
_TCOL = 4096


def _repack_body(a_ref, b_ref, o_ref):
    ii = lax.broadcasted_iota(jnp.int32, (_D, _D), 0)
    jj = lax.broadcasted_iota(jnp.int32, (_D, _D), 1)
    ey = (ii == jj).astype(jnp.bfloat16)
    dn = (((0,), (0,)), ((), ()))
    at = lax.dot_general(a_ref[...].astype(jnp.bfloat16), ey, dn,
                         preferred_element_type=jnp.float32)
    bt = lax.dot_general(b_ref[...].astype(jnp.bfloat16), ey, dn,
                         preferred_element_type=jnp.float32)
    o_ref[...] = jnp.concatenate([at, bt], axis=1)


def _repack(utab_t, itab_t):
    grid = (_N + _TCOL - 1) // _TCOL
    return pl.pallas_call(
        _repack_body,
        grid=(grid,),
        in_specs=[
            pl.BlockSpec((_D, _TCOL), lambda i: (0, i)),
            pl.BlockSpec((_D, _TCOL), lambda i: (0, i)),
        ],
        out_specs=pl.BlockSpec((_TCOL, 2 * _D), lambda i: (i, 0)),
        out_shape=jax.ShapeDtypeStruct((_N, 2 * _D), jnp.float32),
    )(utab_t, itab_t)


def _make_sc_gather():
    info = plsc.get_sparse_core_info()
    nc, ns = info.num_cores, info.num_subcores
    nw = nc * ns
    b_per_w = _B // nw
    n_chunks = b_per_w // _IDX_CHUNK
    mesh = plsc.VectorSubcoreMesh(core_axis_name="c", subcore_axis_name="s")

    @functools.partial(
        pl.kernel,
        out_type=(
            jax.ShapeDtypeStruct((_B, 2 * _D), jnp.float32),
            jax.ShapeDtypeStruct((_B, 2 * _D), jnp.float32),
        ),
        mesh=mesh,
        compiler_params=pltpu.CompilerParams(use_tc_tiling_on_sc=True),
        scratch_types=[
            pltpu.VMEM((n_chunks, _IDX_CHUNK), jnp.int32),
            pltpu.VMEM((n_chunks, _IDX_CHUNK), jnp.int32),
            pltpu.VMEM((b_per_w, 2 * _D), jnp.float32),
            pltpu.SemaphoreType.DMA,
        ],
    )
    def gather_kernel(uid_hbm, iid_hbm, tab_hbm, uout_hbm, iout_hbm, uidx_v,
                      iidx_v, rows_v, sem):
        wid = lax.axis_index("s") * nc + lax.axis_index("c")
        base = wid * b_per_w
        for j in range(n_chunks):
            pltpu.sync_copy(uid_hbm.at[pl.ds(base + j * _IDX_CHUNK,
                                             _IDX_CHUNK)], uidx_v.at[j])
            pltpu.sync_copy(iid_hbm.at[pl.ds(base + j * _IDX_CHUNK,
                                             _IDX_CHUNK)], iidx_v.at[j])
        for idx_v, out_hbm in ((uidx_v, uout_hbm), (iidx_v, iout_hbm)):
            copies = []
            for j in range(n_chunks):
                copies.append(pltpu.async_copy(
                    tab_hbm.at[idx_v.at[j]],
                    rows_v.at[pl.ds(j * _IDX_CHUNK, _IDX_CHUNK), :], sem))
            for c in copies:
                c.wait()
            pltpu.sync_copy(rows_v, out_hbm.at[pl.ds(base, b_per_w)])

    return gather_kernel


_sc_gather_cache = None


def _sc_gather(*args):
    global _sc_gather_cache
    if _sc_gather_cache is None:
        _sc_gather_cache = _make_sc_gather()
    return _sc_gather_cache(*args)


_BLK = 512


def _mlp_body(ur_ref, ir_ref, w1u_ref, w1v_ref, b1_ref, w2_ref, b2_ref,
              w3_ref, b3_ref, o_ref):
    u = ur_ref[:, :_D]
    v = ir_ref[:, _D:]
    h = jnp.dot(u, w1u_ref[...], preferred_element_type=jnp.float32)
    h = h + jnp.dot(v, w1v_ref[...], preferred_element_type=jnp.float32)
    h = jnp.maximum(h + b1_ref[...], 0.0)
    h2 = jnp.dot(h, w2_ref[...], preferred_element_type=jnp.float32)
    h2 = jnp.maximum(h2 + b2_ref[...], 0.0)
    o = jnp.sum(h2 * w3_ref[...], axis=1, keepdims=True) + b3_ref[...]
    o_ref[...] = jax.nn.sigmoid(o)


def _mlp(ur, ir, w1u, w1v, b1, w2t, b2, w3r, b3):
    grid = _B // _BLK
    full = lambda i: (0, 0)
    return pl.pallas_call(
        _mlp_body,
        grid=(grid,),
        in_specs=[
            pl.BlockSpec((_BLK, 2 * _D), lambda i: (i, 0)),
            pl.BlockSpec((_BLK, 2 * _D), lambda i: (i, 0)),
            pl.BlockSpec((_D, 256), full),
            pl.BlockSpec((_D, 256), full),
            pl.BlockSpec((1, 256), full),
            pl.BlockSpec((256, 128), full),
            pl.BlockSpec((1, 128), full),
            pl.BlockSpec((1, 128), full),
            pl.BlockSpec((1, 1), full),
        ],
        out_specs=pl.BlockSpec((_BLK, 1), lambda i: (i, 0)),
        out_shape=jax.ShapeDtypeStruct((_B, 1), jnp.float32),
    )(ur, ir, w1u, w1v, b1, w2t, b2, w3r, b3)


def kernel(user, item, user_table, item_table, W1, b1, W2, b2, W3, b3):
    packed = _repack(user_table.T, item_table.T)
    u_rows, i_rows = _sc_gather(user, item, packed)
    w1u = W1[:, :_D].T
    w1v = W1[:, _D:].T
    out = _mlp(u_rows, i_rows, w1u, w1v, b1.reshape(1, 256), W2.T,
               b2.reshape(1, 128), W3, b3.reshape(1, 1))
    return out[:, 0]


# speedup vs baseline: 2.3482x; 1.0397x over previous
import functools

import jax
import jax.numpy as jnp
from jax import lax
from jax.experimental import pallas as pl
from jax.experimental.pallas import tpu as pltpu
from jax.experimental.pallas import tpu_sc as plsc

_B = 16384
_D = 64
_N = 1000000
_IDX_CHUNK = 128
_TCOL = 4096


def _repack_body(a_ref, b_ref, o_ref):
    ii = lax.broadcasted_iota(jnp.int32, (_D, _D), 0)
    jj = lax.broadcasted_iota(jnp.int32, (_D, _D), 1)
    ey = (ii == jj).astype(jnp.bfloat16)
    dn = (((0,), (0,)), ((), ()))
    at = lax.dot_general(a_ref[...].astype(jnp.bfloat16), ey, dn,
                         preferred_element_type=jnp.float32)
    bt = lax.dot_general(b_ref[...].astype(jnp.bfloat16), ey, dn,
                         preferred_element_type=jnp.float32)
    y = jnp.concatenate([at, bt], axis=1)
    be = lax.bitcast_convert_type(y[:_TCOL // 2, :], jnp.uint32)
    bo = lax.bitcast_convert_type(y[_TCOL // 2:, :], jnp.uint32)
    lo = lax.shift_right_logical(be + jnp.uint32(0x8000), jnp.uint32(16))
    hi = (bo + jnp.uint32(0x8000)) & jnp.uint32(0xFFFF0000)
    o_ref[...] = lax.bitcast_convert_type(lo | hi, jnp.float32)


_GRID = (_N + _TCOL - 1) // _TCOL
_PROWS = _GRID * (_TCOL // 2)


def _repack(utab_t, itab_t):
    grid = _GRID
    return pl.pallas_call(
        _repack_body,
        grid=(grid,),
        in_specs=[
            pl.BlockSpec((_D, _TCOL), lambda i: (0, i)),
            pl.BlockSpec((_D, _TCOL), lambda i: (0, i)),
        ],
        out_specs=pl.BlockSpec((_TCOL // 2, 2 * _D), lambda i: (i, 0)),
        out_shape=jax.ShapeDtypeStruct((_PROWS, 2 * _D), jnp.float32),
        compiler_params=pltpu.CompilerParams(
            fuse_transposed_lhs_in_matmul=True),
    )(utab_t, itab_t)


def _make_sc_gather():
    info = plsc.get_sparse_core_info()
    nc, ns = info.num_cores, info.num_subcores
    nw = nc * ns
    b_per_w = _B // nw
    n_chunks = b_per_w // _IDX_CHUNK
    mesh = plsc.VectorSubcoreMesh(core_axis_name="c", subcore_axis_name="s")

    @functools.partial(
        pl.kernel,
        out_type=(
            jax.ShapeDtypeStruct((_B, 2 * _D), jnp.float32),
            jax.ShapeDtypeStruct((_B, 2 * _D), jnp.float32),
        ),
        mesh=mesh,
        compiler_params=pltpu.CompilerParams(use_tc_tiling_on_sc=True),
        scratch_types=[
            pltpu.VMEM((n_chunks, _IDX_CHUNK), jnp.int32),
            pltpu.VMEM((n_chunks, _IDX_CHUNK), jnp.int32),
            pltpu.VMEM((b_per_w, 2 * _D), jnp.float32),
            pltpu.SemaphoreType.DMA,
        ],
    )
    def gather_kernel(uid_hbm, iid_hbm, tab_hbm, uout_hbm, iout_hbm, uidx_v,
                      iidx_v, rows_v, sem):
        wid = lax.axis_index("s") * nc + lax.axis_index("c")
        base = wid * b_per_w
        for j in range(n_chunks):
            pltpu.sync_copy(uid_hbm.at[pl.ds(base + j * _IDX_CHUNK,
                                             _IDX_CHUNK)], uidx_v.at[j])
            pltpu.sync_copy(iid_hbm.at[pl.ds(base + j * _IDX_CHUNK,
                                             _IDX_CHUNK)], iidx_v.at[j])
        for idx_v, out_hbm in ((uidx_v, uout_hbm), (iidx_v, iout_hbm)):
            copies = []
            for j in range(n_chunks):
                copies.append(pltpu.async_copy(
                    tab_hbm.at[idx_v.at[j]],
                    rows_v.at[pl.ds(j * _IDX_CHUNK, _IDX_CHUNK), :], sem))
            for c in copies:
                c.wait()
            pltpu.sync_copy(rows_v, out_hbm.at[pl.ds(base, b_per_w)])

    return gather_kernel


_sc_gather_cache = None


def _sc_gather(*args):
    global _sc_gather_cache
    if _sc_gather_cache is None:
        _sc_gather_cache = _make_sc_gather()
    return _sc_gather_cache(*args)


_BLK = 512


def _mlp_body(ur_ref, ir_ref, pu_ref, pi_ref, w1u_ref, w1v_ref, b1_ref,
              w2_ref, b2_ref, w3_ref, b3_ref, o_ref):
    uw = lax.bitcast_convert_type(ur_ref[...], jnp.uint32)
    iw = lax.bitcast_convert_type(ir_ref[...], jnp.uint32)
    m16 = jnp.uint32(0xFFFF0000)
    u_even = lax.bitcast_convert_type(
        lax.shift_left(uw, jnp.uint32(16)), jnp.float32)
    u_odd = lax.bitcast_convert_type(uw & m16, jnp.float32)
    i_even = lax.bitcast_convert_type(
        lax.shift_left(iw, jnp.uint32(16)), jnp.float32)
    i_odd = lax.bitcast_convert_type(iw & m16, jnp.float32)
    sel_u = jnp.where(pu_ref[...] > 0, u_odd, u_even)
    sel_i = jnp.where(pi_ref[...] > 0, i_odd, i_even)
    u = sel_u[:, :_D]
    v = sel_i[:, _D:]
    h = jnp.dot(u, w1u_ref[...], preferred_element_type=jnp.float32)
    h = h + jnp.dot(v, w1v_ref[...], preferred_element_type=jnp.float32)
    h = jnp.maximum(h + b1_ref[...], 0.0)
    h2 = jnp.dot(h, w2_ref[...], preferred_element_type=jnp.float32)
    h2 = jnp.maximum(h2 + b2_ref[...], 0.0)
    o = jnp.sum(h2 * w3_ref[...], axis=1, keepdims=True) + b3_ref[...]
    o_ref[...] = jax.nn.sigmoid(o)


def _mlp(ur, ir, pu, pi, w1u, w1v, b1, w2t, b2, w3r, b3):
    grid = _B // _BLK
    full = lambda i: (0, 0)
    return pl.pallas_call(
        _mlp_body,
        grid=(grid,),
        in_specs=[
            pl.BlockSpec((_BLK, 2 * _D), lambda i: (i, 0)),
            pl.BlockSpec((_BLK, 2 * _D), lambda i: (i, 0)),
            pl.BlockSpec((_BLK, 2 * _D), lambda i: (i, 0)),
            pl.BlockSpec((_BLK, 2 * _D), lambda i: (i, 0)),
            pl.BlockSpec((_D, 256), full),
            pl.BlockSpec((_D, 256), full),
            pl.BlockSpec((1, 256), full),
            pl.BlockSpec((256, 128), full),
            pl.BlockSpec((1, 128), full),
            pl.BlockSpec((1, 128), full),
            pl.BlockSpec((1, 1), full),
        ],
        out_specs=pl.BlockSpec((_BLK, 1), lambda i: (i, 0)),
        out_shape=jax.ShapeDtypeStruct((_B, 1), jnp.float32),
    )(ur, ir, pu, pi, w1u, w1v, b1, w2t, b2, w3r, b3)


def kernel(user, item, user_table, item_table, W1, b1, W2, b2, W3, b3):
    packed = _repack(user_table.T, item_table.T)
    half = _TCOL // 2
    up = (user >> 12) * half + (user & (half - 1))
    ip = (item >> 12) * half + (item & (half - 1))
    pu = jnp.broadcast_to(((user >> 11) & 1).astype(jnp.float32)[:, None],
                          (_B, 2 * _D))
    pi = jnp.broadcast_to(((item >> 11) & 1).astype(jnp.float32)[:, None],
                          (_B, 2 * _D))
    u_rows, i_rows = _sc_gather(up, ip, packed)
    w1u = W1[:, :_D].T
    w1v = W1[:, _D:].T
    out = _mlp(u_rows, i_rows, pu, pi, w1u, w1v, b1.reshape(1, 256), W2.T,
               b2.reshape(1, 128), W3, b3.reshape(1, 1))
    return out[:, 0]


# speedup vs baseline: 2.7455x; 1.1692x over previous
import functools

import jax
import jax.numpy as jnp
from jax import lax
from jax.experimental import pallas as pl
from jax.experimental.pallas import tpu as pltpu
from jax.experimental.pallas import tpu_sc as plsc

_B = 16384
_D = 64
_N = 1000000
_IDX_CHUNK = 128
_TCOL = 8192
_TSH = _TCOL.bit_length() - 1


def _repack_body(a_ref, b_ref, o_ref):
    ii = lax.broadcasted_iota(jnp.int32, (_D, _D), 0)
    jj = lax.broadcasted_iota(jnp.int32, (_D, _D), 1)
    ey = (ii == jj).astype(jnp.bfloat16)
    dn = (((0,), (0,)), ((), ()))
    at = lax.dot_general(a_ref[...].astype(jnp.bfloat16), ey, dn,
                         preferred_element_type=jnp.float32)
    bt = lax.dot_general(b_ref[...].astype(jnp.bfloat16), ey, dn,
                         preferred_element_type=jnp.float32)
    y = jnp.concatenate([at, bt], axis=1)
    be = lax.bitcast_convert_type(y[:_TCOL // 2, :], jnp.uint32)
    bo = lax.bitcast_convert_type(y[_TCOL // 2:, :], jnp.uint32)
    lo = lax.shift_right_logical(be + jnp.uint32(0x8000), jnp.uint32(16))
    hi = (bo + jnp.uint32(0x8000)) & jnp.uint32(0xFFFF0000)
    o_ref[...] = lax.bitcast_convert_type(lo | hi, jnp.float32)


_GRID = (_N + _TCOL - 1) // _TCOL
_PROWS = _GRID * (_TCOL // 2)


def _repack(utab_t, itab_t):
    grid = _GRID
    return pl.pallas_call(
        _repack_body,
        grid=(grid,),
        in_specs=[
            pl.BlockSpec((_D, _TCOL), lambda i: (0, i)),
            pl.BlockSpec((_D, _TCOL), lambda i: (0, i)),
        ],
        out_specs=pl.BlockSpec((_TCOL // 2, 2 * _D), lambda i: (i, 0)),
        out_shape=jax.ShapeDtypeStruct((_PROWS, 2 * _D), jnp.float32),
        compiler_params=pltpu.CompilerParams(
            fuse_transposed_lhs_in_matmul=True),
    )(utab_t, itab_t)


def _make_sc_gather():
    info = plsc.get_sparse_core_info()
    nc, ns = info.num_cores, info.num_subcores
    nw = nc * ns
    b_per_w = _B // nw
    n_chunks = b_per_w // _IDX_CHUNK
    mesh = plsc.VectorSubcoreMesh(core_axis_name="c", subcore_axis_name="s")

    @functools.partial(
        pl.kernel,
        out_type=(
            jax.ShapeDtypeStruct((_B, 2 * _D), jnp.float32),
            jax.ShapeDtypeStruct((_B, 2 * _D), jnp.float32),
        ),
        mesh=mesh,
        compiler_params=pltpu.CompilerParams(use_tc_tiling_on_sc=True),
        scratch_types=[
            pltpu.VMEM((n_chunks, _IDX_CHUNK), jnp.int32),
            pltpu.VMEM((n_chunks, _IDX_CHUNK), jnp.int32),
            pltpu.VMEM((b_per_w, 2 * _D), jnp.float32),
            pltpu.SemaphoreType.DMA,
        ],
    )
    def gather_kernel(uid_hbm, iid_hbm, tab_hbm, uout_hbm, iout_hbm, uidx_v,
                      iidx_v, rows_v, sem):
        wid = lax.axis_index("s") * nc + lax.axis_index("c")
        base = wid * b_per_w
        for j in range(n_chunks):
            pltpu.sync_copy(uid_hbm.at[pl.ds(base + j * _IDX_CHUNK,
                                             _IDX_CHUNK)], uidx_v.at[j])
            pltpu.sync_copy(iid_hbm.at[pl.ds(base + j * _IDX_CHUNK,
                                             _IDX_CHUNK)], iidx_v.at[j])
        for idx_v, out_hbm in ((uidx_v, uout_hbm), (iidx_v, iout_hbm)):
            copies = []
            for j in range(n_chunks):
                copies.append(pltpu.async_copy(
                    tab_hbm.at[idx_v.at[j]],
                    rows_v.at[pl.ds(j * _IDX_CHUNK, _IDX_CHUNK), :], sem))
            for c in copies:
                c.wait()
            pltpu.sync_copy(rows_v, out_hbm.at[pl.ds(base, b_per_w)])

    return gather_kernel


_sc_gather_cache = None


def _sc_gather(*args):
    global _sc_gather_cache
    if _sc_gather_cache is None:
        _sc_gather_cache = _make_sc_gather()
    return _sc_gather_cache(*args)


_BLK = 512


def _mlp_body(ur_ref, ir_ref, pu_ref, pi_ref, w1u_ref, w1v_ref, b1_ref,
              w2_ref, b2_ref, w3_ref, b3_ref, o_ref):
    uw = lax.bitcast_convert_type(ur_ref[...], jnp.uint32)
    iw = lax.bitcast_convert_type(ir_ref[...], jnp.uint32)
    m16 = jnp.uint32(0xFFFF0000)
    u_even = lax.bitcast_convert_type(
        lax.shift_left(uw, jnp.uint32(16)), jnp.float32)
    u_odd = lax.bitcast_convert_type(uw & m16, jnp.float32)
    i_even = lax.bitcast_convert_type(
        lax.shift_left(iw, jnp.uint32(16)), jnp.float32)
    i_odd = lax.bitcast_convert_type(iw & m16, jnp.float32)
    sel_u = jnp.where(pu_ref[...] > 0, u_odd, u_even)
    sel_i = jnp.where(pi_ref[...] > 0, i_odd, i_even)
    u = sel_u[:, :_D]
    v = sel_i[:, _D:]
    h = jnp.dot(u, w1u_ref[...], preferred_element_type=jnp.float32)
    h = h + jnp.dot(v, w1v_ref[...], preferred_element_type=jnp.float32)
    h = jnp.maximum(h + b1_ref[...], 0.0)
    h2 = jnp.dot(h, w2_ref[...], preferred_element_type=jnp.float32)
    h2 = jnp.maximum(h2 + b2_ref[...], 0.0)
    o = jnp.sum(h2 * w3_ref[...], axis=1, keepdims=True) + b3_ref[...]
    o_ref[...] = jax.nn.sigmoid(o)


def _mlp(ur, ir, pu, pi, w1u, w1v, b1, w2t, b2, w3r, b3):
    grid = _B // _BLK
    full = lambda i: (0, 0)
    return pl.pallas_call(
        _mlp_body,
        grid=(grid,),
        in_specs=[
            pl.BlockSpec((_BLK, 2 * _D), lambda i: (i, 0)),
            pl.BlockSpec((_BLK, 2 * _D), lambda i: (i, 0)),
            pl.BlockSpec((_BLK, 2 * _D), lambda i: (i, 0)),
            pl.BlockSpec((_BLK, 2 * _D), lambda i: (i, 0)),
            pl.BlockSpec((_D, 256), full),
            pl.BlockSpec((_D, 256), full),
            pl.BlockSpec((1, 256), full),
            pl.BlockSpec((256, 128), full),
            pl.BlockSpec((1, 128), full),
            pl.BlockSpec((1, 128), full),
            pl.BlockSpec((1, 1), full),
        ],
        out_specs=pl.BlockSpec((_BLK, 1), lambda i: (i, 0)),
        out_shape=jax.ShapeDtypeStruct((_B, 1), jnp.float32),
    )(ur, ir, pu, pi, w1u, w1v, b1, w2t, b2, w3r, b3)


def kernel(user, item, user_table, item_table, W1, b1, W2, b2, W3, b3):
    packed = _repack(user_table.T, item_table.T)
    half = _TCOL // 2
    up = (user >> _TSH) * half + (user & (half - 1))
    ip = (item >> _TSH) * half + (item & (half - 1))
    pu = jnp.broadcast_to(
        ((user >> (_TSH - 1)) & 1).astype(jnp.float32)[:, None],
        (_B, 2 * _D))
    pi = jnp.broadcast_to(
        ((item >> (_TSH - 1)) & 1).astype(jnp.float32)[:, None],
        (_B, 2 * _D))
    u_rows, i_rows = _sc_gather(up, ip, packed)
    w1u = W1[:, :_D].T
    w1v = W1[:, _D:].T
    out = _mlp(u_rows, i_rows, pu, pi, w1u, w1v, b1.reshape(1, 256), W2.T,
               b2.reshape(1, 128), W3, b3.reshape(1, 1))
    return out[:, 0]


# speedup vs baseline: 3.1127x; 1.1337x over previous
import functools

import jax
import jax.numpy as jnp
from jax import lax
from jax.experimental import pallas as pl
from jax.experimental.pallas import tpu as pltpu
from jax.experimental.pallas import tpu_sc as plsc

_B = 16384
_D = 64
_N = 1000000
_IDX_CHUNK = 128
_TCOL = 16384
_TSH = _TCOL.bit_length() - 1


def _repack_body(a_ref, b_ref, o_ref):
    ii = lax.broadcasted_iota(jnp.int32, (_D, _D), 0)
    jj = lax.broadcasted_iota(jnp.int32, (_D, _D), 1)
    ey = (ii == jj).astype(jnp.bfloat16)
    dn = (((0,), (0,)), ((), ()))
    at = lax.dot_general(a_ref[...].astype(jnp.bfloat16), ey, dn,
                         preferred_element_type=jnp.float32)
    bt = lax.dot_general(b_ref[...].astype(jnp.bfloat16), ey, dn,
                         preferred_element_type=jnp.float32)
    y = jnp.concatenate([at, bt], axis=1)
    be = lax.bitcast_convert_type(y[:_TCOL // 2, :], jnp.uint32)
    bo = lax.bitcast_convert_type(y[_TCOL // 2:, :], jnp.uint32)
    lo = lax.shift_right_logical(be + jnp.uint32(0x8000), jnp.uint32(16))
    hi = (bo + jnp.uint32(0x8000)) & jnp.uint32(0xFFFF0000)
    o_ref[...] = lax.bitcast_convert_type(lo | hi, jnp.float32)


_GRID = (_N + _TCOL - 1) // _TCOL
_PROWS = _GRID * (_TCOL // 2)


def _repack(utab_t, itab_t):
    grid = _GRID
    return pl.pallas_call(
        _repack_body,
        grid=(grid,),
        in_specs=[
            pl.BlockSpec((_D, _TCOL), lambda i: (0, i)),
            pl.BlockSpec((_D, _TCOL), lambda i: (0, i)),
        ],
        out_specs=pl.BlockSpec((_TCOL // 2, 2 * _D), lambda i: (i, 0)),
        out_shape=jax.ShapeDtypeStruct((_PROWS, 2 * _D), jnp.float32),
        compiler_params=pltpu.CompilerParams(
            fuse_transposed_lhs_in_matmul=True),
    )(utab_t, itab_t)


def _make_sc_gather():
    info = plsc.get_sparse_core_info()
    nc, ns = info.num_cores, info.num_subcores
    nw = nc * ns
    b_per_w = _B // nw
    n_chunks = b_per_w // _IDX_CHUNK
    mesh = plsc.VectorSubcoreMesh(core_axis_name="c", subcore_axis_name="s")

    @functools.partial(
        pl.kernel,
        out_type=(
            jax.ShapeDtypeStruct((_B, 2 * _D), jnp.float32),
            jax.ShapeDtypeStruct((_B, 2 * _D), jnp.float32),
        ),
        mesh=mesh,
        compiler_params=pltpu.CompilerParams(use_tc_tiling_on_sc=True),
        scratch_types=[
            pltpu.VMEM((n_chunks, _IDX_CHUNK), jnp.int32),
            pltpu.VMEM((n_chunks, _IDX_CHUNK), jnp.int32),
            pltpu.VMEM((b_per_w, 2 * _D), jnp.float32),
            pltpu.SemaphoreType.DMA,
        ],
    )
    def gather_kernel(uid_hbm, iid_hbm, tab_hbm, uout_hbm, iout_hbm, uidx_v,
                      iidx_v, rows_v, sem):
        wid = lax.axis_index("s") * nc + lax.axis_index("c")
        base = wid * b_per_w
        for j in range(n_chunks):
            pltpu.sync_copy(uid_hbm.at[pl.ds(base + j * _IDX_CHUNK,
                                             _IDX_CHUNK)], uidx_v.at[j])
            pltpu.sync_copy(iid_hbm.at[pl.ds(base + j * _IDX_CHUNK,
                                             _IDX_CHUNK)], iidx_v.at[j])
        for idx_v, out_hbm in ((uidx_v, uout_hbm), (iidx_v, iout_hbm)):
            copies = []
            for j in range(n_chunks):
                copies.append(pltpu.async_copy(
                    tab_hbm.at[idx_v.at[j]],
                    rows_v.at[pl.ds(j * _IDX_CHUNK, _IDX_CHUNK), :], sem))
            for c in copies:
                c.wait()
            pltpu.sync_copy(rows_v, out_hbm.at[pl.ds(base, b_per_w)])

    return gather_kernel


_sc_gather_cache = None


def _sc_gather(*args):
    global _sc_gather_cache
    if _sc_gather_cache is None:
        _sc_gather_cache = _make_sc_gather()
    return _sc_gather_cache(*args)


_BLK = 2048


def _mlp_body(ur_ref, ir_ref, pu_ref, pi_ref, w1u_ref, w1v_ref, b1_ref,
              w2_ref, b2_ref, w3_ref, b3_ref, o_ref):
    uw = lax.bitcast_convert_type(ur_ref[...], jnp.uint32)
    iw = lax.bitcast_convert_type(ir_ref[...], jnp.uint32)
    m16 = jnp.uint32(0xFFFF0000)
    u_even = lax.bitcast_convert_type(
        lax.shift_left(uw, jnp.uint32(16)), jnp.float32)
    u_odd = lax.bitcast_convert_type(uw & m16, jnp.float32)
    i_even = lax.bitcast_convert_type(
        lax.shift_left(iw, jnp.uint32(16)), jnp.float32)
    i_odd = lax.bitcast_convert_type(iw & m16, jnp.float32)
    sel_u = jnp.where(pu_ref[...] > 0, u_odd, u_even)
    sel_i = jnp.where(pi_ref[...] > 0, i_odd, i_even)
    u = sel_u[:, :_D]
    v = sel_i[:, _D:]
    h = jnp.dot(u, w1u_ref[...], preferred_element_type=jnp.float32)
    h = h + jnp.dot(v, w1v_ref[...], preferred_element_type=jnp.float32)
    h = jnp.maximum(h + b1_ref[...], 0.0)
    h2 = jnp.dot(h, w2_ref[...], preferred_element_type=jnp.float32)
    h2 = jnp.maximum(h2 + b2_ref[...], 0.0)
    o = jnp.sum(h2 * w3_ref[...], axis=1, keepdims=True) + b3_ref[...]
    o_ref[...] = jax.nn.sigmoid(o)


def _mlp(ur, ir, pu, pi, w1u, w1v, b1, w2t, b2, w3r, b3):
    grid = _B // _BLK
    full = lambda i: (0, 0)
    return pl.pallas_call(
        _mlp_body,
        grid=(grid,),
        in_specs=[
            pl.BlockSpec((_BLK, 2 * _D), lambda i: (i, 0)),
            pl.BlockSpec((_BLK, 2 * _D), lambda i: (i, 0)),
            pl.BlockSpec((_BLK, 2 * _D), lambda i: (i, 0)),
            pl.BlockSpec((_BLK, 2 * _D), lambda i: (i, 0)),
            pl.BlockSpec((_D, 256), full),
            pl.BlockSpec((_D, 256), full),
            pl.BlockSpec((1, 256), full),
            pl.BlockSpec((256, 128), full),
            pl.BlockSpec((1, 128), full),
            pl.BlockSpec((1, 128), full),
            pl.BlockSpec((1, 1), full),
        ],
        out_specs=pl.BlockSpec((_BLK, 1), lambda i: (i, 0)),
        out_shape=jax.ShapeDtypeStruct((_B, 1), jnp.float32),
    )(ur, ir, pu, pi, w1u, w1v, b1, w2t, b2, w3r, b3)


def kernel(user, item, user_table, item_table, W1, b1, W2, b2, W3, b3):
    packed = _repack(user_table.T, item_table.T)
    half = _TCOL // 2
    up = (user >> _TSH) * half + (user & (half - 1))
    ip = (item >> _TSH) * half + (item & (half - 1))
    pu = jnp.broadcast_to(
        ((user >> (_TSH - 1)) & 1).astype(jnp.float32)[:, None],
        (_B, 2 * _D))
    pi = jnp.broadcast_to(
        ((item >> (_TSH - 1)) & 1).astype(jnp.float32)[:, None],
        (_B, 2 * _D))
    u_rows, i_rows = _sc_gather(up, ip, packed)
    w1u = W1[:, :_D].T
    w1v = W1[:, _D:].T
    out = _mlp(u_rows, i_rows, pu, pi, w1u, w1v, b1.reshape(1, 256), W2.T,
               b2.reshape(1, 128), W3, b3.reshape(1, 1))
    return out[:, 0]


# speedup vs baseline: 3.1927x; 1.0257x over previous
import functools

import jax
import jax.numpy as jnp
from jax import lax
from jax.experimental import pallas as pl
from jax.experimental.pallas import tpu as pltpu
from jax.experimental.pallas import tpu_sc as plsc

_B = 16384
_D = 64
_N = 1000000
_IDX_CHUNK = 128
_TCOL = 16384
_TSH = _TCOL.bit_length() - 1


def _repack_body(a_ref, b_ref, o_ref):
    ii = lax.broadcasted_iota(jnp.int32, (_D, _D), 0)
    jj = lax.broadcasted_iota(jnp.int32, (_D, _D), 1)
    ey = (ii == jj).astype(jnp.bfloat16)
    dn = (((0,), (0,)), ((), ()))
    at = lax.dot_general(a_ref[...].astype(jnp.bfloat16), ey, dn,
                         preferred_element_type=jnp.float32)
    bt = lax.dot_general(b_ref[...].astype(jnp.bfloat16), ey, dn,
                         preferred_element_type=jnp.float32)
    y = jnp.concatenate([at, bt], axis=1)
    be = lax.bitcast_convert_type(y[:_TCOL // 2, :], jnp.uint32)
    bo = lax.bitcast_convert_type(y[_TCOL // 2:, :], jnp.uint32)
    lo = lax.shift_right_logical(be + jnp.uint32(0x8000), jnp.uint32(16))
    hi = (bo + jnp.uint32(0x8000)) & jnp.uint32(0xFFFF0000)
    o_ref[...] = lax.bitcast_convert_type(lo | hi, jnp.float32)


_GRID = (_N + _TCOL - 1) // _TCOL
_PROWS = _GRID * (_TCOL // 2)


def _repack(utab_t, itab_t):
    grid = _GRID
    return pl.pallas_call(
        _repack_body,
        grid=(grid,),
        in_specs=[
            pl.BlockSpec((_D, _TCOL), lambda i: (0, i)),
            pl.BlockSpec((_D, _TCOL), lambda i: (0, i)),
        ],
        out_specs=pl.BlockSpec((_TCOL // 2, 2 * _D), lambda i: (i, 0)),
        out_shape=jax.ShapeDtypeStruct((_PROWS, 2 * _D), jnp.float32),
        compiler_params=pltpu.CompilerParams(
            fuse_transposed_lhs_in_matmul=True),
    )(utab_t, itab_t)


def _make_sc_gather():
    info = plsc.get_sparse_core_info()
    nc, ns = info.num_cores, info.num_subcores
    nw = nc * ns
    b_per_w = _B // nw
    n_chunks = b_per_w // _IDX_CHUNK
    mesh = plsc.VectorSubcoreMesh(core_axis_name="c", subcore_axis_name="s")

    @functools.partial(
        pl.kernel,
        out_type=(
            jax.ShapeDtypeStruct((_B, 2 * _D), jnp.float32),
            jax.ShapeDtypeStruct((_B, 2 * _D), jnp.float32),
        ),
        mesh=mesh,
        compiler_params=pltpu.CompilerParams(use_tc_tiling_on_sc=True),
        scratch_types=[
            pltpu.VMEM((n_chunks, _IDX_CHUNK), jnp.int32),
            pltpu.VMEM((n_chunks, _IDX_CHUNK), jnp.int32),
            pltpu.VMEM((b_per_w, 2 * _D), jnp.float32),
            pltpu.SemaphoreType.DMA,
        ],
    )
    def gather_kernel(uid_hbm, iid_hbm, tab_hbm, uout_hbm, iout_hbm, uidx_v,
                      iidx_v, rows_v, sem):
        wid = lax.axis_index("s") * nc + lax.axis_index("c")
        base = wid * b_per_w
        for j in range(n_chunks):
            pltpu.sync_copy(uid_hbm.at[pl.ds(base + j * _IDX_CHUNK,
                                             _IDX_CHUNK)], uidx_v.at[j])
            pltpu.sync_copy(iid_hbm.at[pl.ds(base + j * _IDX_CHUNK,
                                             _IDX_CHUNK)], iidx_v.at[j])
        for idx_v, out_hbm in ((uidx_v, uout_hbm), (iidx_v, iout_hbm)):
            copies = []
            for j in range(n_chunks):
                copies.append(pltpu.async_copy(
                    tab_hbm.at[idx_v.at[j]],
                    rows_v.at[pl.ds(j * _IDX_CHUNK, _IDX_CHUNK), :], sem))
            for c in copies:
                c.wait()
            pltpu.sync_copy(rows_v, out_hbm.at[pl.ds(base, b_per_w)])

    return gather_kernel


_sc_gather_cache = None


def _sc_gather(*args):
    global _sc_gather_cache
    if _sc_gather_cache is None:
        _sc_gather_cache = _make_sc_gather()
    return _sc_gather_cache(*args)


_BLK = 2048


def _mlp_body(ur_ref, ir_ref, pu_ref, pi_ref, w1u_ref, w1v_ref, b1_ref,
              w2_ref, b2_ref, w3_ref, b3_ref, o_ref):
    uw = lax.bitcast_convert_type(ur_ref[...], jnp.uint32)
    iw = lax.bitcast_convert_type(ir_ref[...], jnp.uint32)
    m16 = jnp.uint32(0xFFFF0000)
    u_even = lax.bitcast_convert_type(
        lax.shift_left(uw, jnp.uint32(16)), jnp.float32)
    u_odd = lax.bitcast_convert_type(uw & m16, jnp.float32)
    i_even = lax.bitcast_convert_type(
        lax.shift_left(iw, jnp.uint32(16)), jnp.float32)
    i_odd = lax.bitcast_convert_type(iw & m16, jnp.float32)
    sel_u = jnp.where(pu_ref[...] > 0, u_odd, u_even)
    sel_i = jnp.where(pi_ref[...] > 0, i_odd, i_even)
    u = sel_u[:, :_D]
    v = sel_i[:, _D:]
    h = jnp.dot(u, w1u_ref[...], preferred_element_type=jnp.float32)
    h = h + jnp.dot(v, w1v_ref[...], preferred_element_type=jnp.float32)
    h = jnp.maximum(h + b1_ref[...], 0.0)
    h2 = jnp.dot(h, w2_ref[...], preferred_element_type=jnp.float32)
    h2 = jnp.maximum(h2 + b2_ref[...], 0.0)
    o = jnp.sum(h2 * w3_ref[...], axis=1, keepdims=True) + b3_ref[...]
    o_ref[...] = jax.nn.sigmoid(o)


def _mlp(ur, ir, pu, pi, w1u, w1v, b1, w2t, b2, w3r, b3):
    grid = _B // _BLK
    full = lambda i: (0, 0)
    return pl.pallas_call(
        _mlp_body,
        grid=(grid,),
        in_specs=[
            pl.BlockSpec((_BLK, 2 * _D), lambda i: (i, 0)),
            pl.BlockSpec((_BLK, 2 * _D), lambda i: (i, 0)),
            pl.BlockSpec((_BLK, 2 * _D), lambda i: (i, 0)),
            pl.BlockSpec((_BLK, 2 * _D), lambda i: (i, 0)),
            pl.BlockSpec((_D, 256), full),
            pl.BlockSpec((_D, 256), full),
            pl.BlockSpec((1, 256), full),
            pl.BlockSpec((256, 128), full),
            pl.BlockSpec((1, 128), full),
            pl.BlockSpec((1, 128), full),
            pl.BlockSpec((1, 1), full),
        ],
        out_specs=pl.BlockSpec((_BLK, 1), lambda i: (i, 0)),
        out_shape=jax.ShapeDtypeStruct((_B, 1), jnp.float32),
    )(ur, ir, pu, pi, w1u, w1v, b1, w2t, b2, w3r, b3)


def kernel(user, item, user_table, item_table, W1, b1, W2, b2, W3, b3):
    packed = _repack(user_table.T, item_table.T)
    half = _TCOL // 2
    up = (user >> _TSH) * half + (user & (half - 1))
    ip = (item >> _TSH) * half + (item & (half - 1))
    pu = jnp.broadcast_to(
        ((user >> (_TSH - 1)) & 1).astype(jnp.bfloat16)[:, None],
        (_B, 2 * _D))
    pi = jnp.broadcast_to(
        ((item >> (_TSH - 1)) & 1).astype(jnp.bfloat16)[:, None],
        (_B, 2 * _D))
    u_rows, i_rows = _sc_gather(up, ip, packed)
    w1u = W1[:, :_D].T
    w1v = W1[:, _D:].T
    out = _mlp(u_rows, i_rows, pu, pi, w1u, w1v, b1.reshape(1, 256), W2.T,
               b2.reshape(1, 128), W3, b3.reshape(1, 1))
    return out[:, 0]
